# baseline (speedup 1.0000x reference)
import functools

import jax
import jax.numpy as jnp
from jax import lax
from jax.experimental import pallas as pl
from jax.experimental.pallas import tpu as pltpu
from jax.experimental.pallas import tpu_sc as plsc

B = 1024
V = 100000

BBLK = 512
VBLK = 8192
NB = B // BBLK
NV = (V + VBLK - 1) // VBLK

NC = 2
NS = 16
NW = NC * NS
RPW = B // NW
CH = 10000
NCHUNK = V // CH
NVREG = CH // 16


def _count_body(x_ref, l_ref, v_ref, out_ref, cg_ref, ce_ref):
    b = pl.program_id(0)
    vv = pl.program_id(1)
    nv = pl.num_programs(1)
    blk = x_ref[...]
    gcol = vv * VBLK + lax.broadcasted_iota(jnp.int32, blk.shape, 1)
    valid = gcol < V
    vb = v_ref[...]
    lb = l_ref[...]
    gt = jnp.where((blk > vb) & valid, 1, 0)
    eqb = jnp.where((blk == vb) & (gcol < lb) & valid, 1, 0)
    cg = jnp.sum(gt, axis=1, keepdims=True)
    ce = jnp.sum(eqb, axis=1, keepdims=True)

    @pl.when(vv == 0)
    def _():
        cg_ref[...] = cg
        ce_ref[...] = ce

    @pl.when(vv > 0)
    def _():
        cg_ref[...] += cg
        ce_ref[...] += ce

    @pl.when(vv == nv - 1)
    def _():
        rank = cg_ref[...] + ce_ref[...]
        scale = jnp.float32(100.0 / B)
        a1 = jnp.sum(jnp.where(rank == 0, scale, 0.0))
        a5 = jnp.sum(jnp.where(rank < 5, scale, 0.0))

        @pl.when(b == 0)
        def _():
            out_ref[0] = a1
            out_ref[1] = a5

        @pl.when(b > 0)
        def _():
            out_ref[0] += a1
            out_ref[1] += a5


_count_call = pl.pallas_call(
    _count_body,
    grid=(NB, NV),
    in_specs=[
        pl.BlockSpec((BBLK, VBLK), lambda b, v: (b, v)),
        pl.BlockSpec((BBLK, 1), lambda b, v: (b, 0)),
        pl.BlockSpec((BBLK, 1), lambda b, v: (b, 0)),
    ],
    out_specs=pl.BlockSpec(memory_space=pltpu.SMEM),
    out_shape=jax.ShapeDtypeStruct((2,), jnp.float32),
    scratch_shapes=[
        pltpu.VMEM((BBLK, 1), jnp.int32),
        pltpu.VMEM((BBLK, 1), jnp.int32),
    ],
)


def _scan_chunk(buf, gbase, m, mi):
    lane = lax.iota(jnp.int32, 16)

    def body(i, carry):
        m, mi = carry
        x = buf[pl.ds(i * 16, 16)]
        idx = gbase + i * 16 + lane
        cmp = x > m
        return jnp.where(cmp, x, m), jnp.where(cmp, idx, mi)

    return lax.fori_loop(0, NVREG, body, (m, mi), unroll=8)


def _sc_body(cls_hbm, enc_hbm, l_hbm, v_hbm, buf0, buf1, idx_v, flat_v,
             vals_v, sem0, sem1, semg):
    wid = lax.axis_index("s") * NC + lax.axis_index("c")
    row0 = wid * RPW
    lane = lax.iota(jnp.int32, 16)

    def do_row(row, acc, slot):
        rbase = row * V
        pltpu.make_async_copy(cls_hbm.at[pl.ds(rbase, CH)], buf0, sem0).start()

        def chunk2(k, carry):
            m, mi = carry
            c0 = 2 * k
            pltpu.make_async_copy(
                cls_hbm.at[pl.ds(rbase + (c0 + 1) * CH, CH)], buf1, sem1
            ).start()
            pltpu.make_async_copy(cls_hbm.at[pl.ds(0, CH)], buf0, sem0).wait()
            m, mi = _scan_chunk(buf0, rbase + c0 * CH, m, mi)

            @pl.when(c0 + 2 < NCHUNK)
            def _():
                pltpu.make_async_copy(
                    cls_hbm.at[pl.ds(rbase + (c0 + 2) * CH, CH)], buf0, sem0
                ).start()

            pltpu.make_async_copy(cls_hbm.at[pl.ds(0, CH)], buf1, sem1).wait()
            m, mi = _scan_chunk(buf1, rbase + (c0 + 1) * CH, m, mi)
            return m, mi

        m0 = jnp.full((16,), -jnp.inf, jnp.float32)
        i0 = jnp.zeros((16,), jnp.int32)
        m, mi = lax.fori_loop(0, NCHUNK // 2, chunk2, (m0, i0))
        for k in (8, 4, 2, 1):
            perm = lane ^ k
            om = m.at[perm].get(mode="promise_in_bounds")
            omi = mi.at[perm].get(mode="promise_in_bounds")
            take = (om > m) | ((om == m) & (omi < mi))
            m = jnp.where(take, om, m)
            mi = jnp.where(take, omi, mi)
        return jnp.where(lane == slot, mi, acc)

    for half in range(RPW // 16):
        def row_body(r, acc, _half=half):
            return do_row(row0 + _half * 16 + r, acc, r)

        acc = lax.fori_loop(0, 16, row_body, jnp.zeros((16,), jnp.int32))
        idx_v[pl.ds(half * 16, 16)] = acc

    for half in range(RPW // 16):
        rows = row0 + half * 16 + lane
        flat_v[pl.ds(half * 16, 16)] = idx_v[pl.ds(half * 16, 16)] + rows * V
    pltpu.async_copy(enc_hbm.at[flat_v], vals_v, semg).wait()
    pltpu.sync_copy(idx_v, l_hbm.at[pl.ds(row0, RPW)])
    pltpu.sync_copy(vals_v, v_hbm.at[pl.ds(row0, RPW)])


@functools.cache
def _sc_argmax_gather():
    return pl.kernel(
        _sc_body,
        mesh=plsc.VectorSubcoreMesh(core_axis_name="c", subcore_axis_name="s"),
        out_type=[
            jax.ShapeDtypeStruct((B,), jnp.int32),
            jax.ShapeDtypeStruct((B,), jnp.float32),
        ],
        scratch_types=[
            pltpu.VMEM((CH,), jnp.float32),
            pltpu.VMEM((CH,), jnp.float32),
            pltpu.VMEM((RPW,), jnp.int32),
            pltpu.VMEM((RPW,), jnp.int32),
            pltpu.VMEM((RPW,), jnp.float32),
            pltpu.SemaphoreType.DMA,
            pltpu.SemaphoreType.DMA,
            pltpu.SemaphoreType.DMA,
        ],
    )


def kernel(enc_score_p0, dec_scores, class_h_target, dec_target):
    labels, v = _sc_argmax_gather()(
        class_h_target.reshape(B * V), enc_score_p0.reshape(B * V)
    )
    return _count_call(enc_score_p0, labels.reshape(B, 1), v.reshape(B, 1))


# speedup vs baseline: 1.7895x; 1.7895x over previous
import functools

import jax
import jax.numpy as jnp
from jax import lax
from jax.experimental import pallas as pl
from jax.experimental.pallas import tpu as pltpu
from jax.experimental.pallas import tpu_sc as plsc

B = 1024
V = 100000

BBLK = 512
VBLK = 8192
NB = B // BBLK
NV = (V + VBLK - 1) // VBLK

NC = 2
NS = 16
NW = NC * NS
RPW = B // NW
VMAIN = 99968
CH = 2048
NFULL = VMAIN // CH
TAIL = VMAIN - NFULL * CH
NVREG = CH // 16
NVREG_TAIL = TAIL // 16
EW = 128


def _count_body(x_ref, l_ref, v_ref, out_ref, cg_ref, ce_ref):
    b = pl.program_id(0)
    vv = pl.program_id(1)
    nv = pl.num_programs(1)
    blk = x_ref[...]
    gcol = vv * VBLK + lax.broadcasted_iota(jnp.int32, blk.shape, 1)
    valid = gcol < V
    vb = v_ref[...]
    lb = l_ref[...]
    gt = jnp.where((blk > vb) & valid, 1, 0)
    eqb = jnp.where((blk == vb) & (gcol < lb) & valid, 1, 0)
    cg = jnp.sum(gt, axis=1, keepdims=True)
    ce = jnp.sum(eqb, axis=1, keepdims=True)

    @pl.when(vv == 0)
    def _():
        cg_ref[...] = cg
        ce_ref[...] = ce

    @pl.when(vv > 0)
    def _():
        cg_ref[...] += cg
        ce_ref[...] += ce

    @pl.when(vv == nv - 1)
    def _():
        rank = cg_ref[...] + ce_ref[...]
        scale = jnp.float32(100.0 / B)
        a1 = jnp.sum(jnp.where(rank == 0, scale, 0.0))
        a5 = jnp.sum(jnp.where(rank < 5, scale, 0.0))

        @pl.when(b == 0)
        def _():
            out_ref[0] = a1
            out_ref[1] = a5

        @pl.when(b > 0)
        def _():
            out_ref[0] += a1
            out_ref[1] += a5


_count_call = pl.pallas_call(
    _count_body,
    grid=(NB, NV),
    in_specs=[
        pl.BlockSpec((BBLK, VBLK), lambda b, v: (b, v)),
        pl.BlockSpec((BBLK, 1), lambda b, v: (b, 0)),
        pl.BlockSpec((BBLK, 1), lambda b, v: (b, 0)),
    ],
    out_specs=pl.BlockSpec(memory_space=pltpu.SMEM),
    out_shape=jax.ShapeDtypeStruct((2,), jnp.float32),
    scratch_shapes=[
        pltpu.VMEM((BBLK, 1), jnp.int32),
        pltpu.VMEM((BBLK, 1), jnp.int32),
    ],
)


def _scan_group2(cbuf, ebuf, cbase, nv, ms, mis, evs):
    lane = lax.iota(jnp.int32, 16)

    def body(i, carry):
        ms, mis, evs = carry
        idx = cbase + i * 16 + lane
        nms, nmis, nevs = [], [], []
        for r in range(8):
            x = cbuf[r, pl.ds(i * 16, 16)]
            y = ebuf[r, pl.ds(i * 16, 16)]
            cmp = x > ms[r]
            nms.append(jnp.where(cmp, x, ms[r]))
            nmis.append(jnp.where(cmp, idx, mis[r]))
            nevs.append(jnp.where(cmp, y, evs[r]))
        return tuple(nms), tuple(nmis), tuple(nevs)

    return lax.fori_loop(0, nv, body, (ms, mis, evs), unroll=2)


def _sc_body(cls_hbm, enc_hbm, ctail_hbm, etail_hbm, l_hbm, v_hbm,
             cbuf0, cbuf1, ebuf0, ebuf1, ctbuf, etbuf, c32, e32,
             idx_v, vals_v,
             semc0, semc1, seme0, seme1, semt, semu, sem32):
    wid = lax.axis_index("s") * NC + lax.axis_index("c")
    row0 = pl.multiple_of(wid * RPW, RPW)
    lane = lax.iota(jnp.int32, 16)

    accl = [jnp.zeros((16,), jnp.int32) for _ in range(RPW // 16)]
    accv = [jnp.zeros((16,), jnp.float32) for _ in range(RPW // 16)]
    for g in range(RPW // 8):
        rs = pl.multiple_of(row0 + 8 * g, 8)
        pltpu.make_async_copy(
            cls_hbm.at[pl.ds(rs, 8), pl.ds(0, CH)], cbuf0, semc0).start()
        pltpu.make_async_copy(
            enc_hbm.at[pl.ds(rs, 8), pl.ds(0, CH)], ebuf0, seme0).start()
        pltpu.make_async_copy(
            cls_hbm.at[pl.ds(rs, 8), pl.ds(NFULL * CH, TAIL)],
            ctbuf, semt).start()
        pltpu.make_async_copy(
            enc_hbm.at[pl.ds(rs, 8), pl.ds(NFULL * CH, TAIL)],
            etbuf, semu).start()
        pltpu.make_async_copy(ctail_hbm.at[pl.ds(rs, 8)], c32, sem32).start()
        pltpu.make_async_copy(etail_hbm.at[pl.ds(rs, 8)], e32, sem32).start()

        def chunk2(k, carry, _rs=rs):
            ms, mis, evs = carry
            c0 = 2 * k
            pltpu.make_async_copy(
                cls_hbm.at[pl.ds(_rs, 8), pl.ds((c0 + 1) * CH, CH)],
                cbuf1, semc1).start()
            pltpu.make_async_copy(
                enc_hbm.at[pl.ds(_rs, 8), pl.ds((c0 + 1) * CH, CH)],
                ebuf1, seme1).start()
            pltpu.make_async_copy(
                cls_hbm.at[pl.ds(_rs, 8), pl.ds(0, CH)], cbuf0, semc0).wait()
            pltpu.make_async_copy(
                enc_hbm.at[pl.ds(_rs, 8), pl.ds(0, CH)], ebuf0, seme0).wait()
            ms, mis, evs = _scan_group2(
                cbuf0, ebuf0, c0 * CH, NVREG, ms, mis, evs)

            @pl.when(c0 + 2 < NFULL)
            def _():
                pltpu.make_async_copy(
                    cls_hbm.at[pl.ds(_rs, 8), pl.ds((c0 + 2) * CH, CH)],
                    cbuf0, semc0).start()
                pltpu.make_async_copy(
                    enc_hbm.at[pl.ds(_rs, 8), pl.ds((c0 + 2) * CH, CH)],
                    ebuf0, seme0).start()

            pltpu.make_async_copy(
                cls_hbm.at[pl.ds(_rs, 8), pl.ds(0, CH)], cbuf1, semc1).wait()
            pltpu.make_async_copy(
                enc_hbm.at[pl.ds(_rs, 8), pl.ds(0, CH)], ebuf1, seme1).wait()
            ms, mis, evs = _scan_group2(
                cbuf1, ebuf1, (c0 + 1) * CH, NVREG, ms, mis, evs)
            return ms, mis, evs

        m0 = tuple(jnp.full((16,), -jnp.inf, jnp.float32) for _ in range(8))
        i0 = tuple(jnp.zeros((16,), jnp.int32) for _ in range(8))
        e0 = tuple(jnp.zeros((16,), jnp.float32) for _ in range(8))
        ms, mis, evs = lax.fori_loop(0, NFULL // 2, chunk2, (m0, i0, e0))
        pltpu.make_async_copy(
            cls_hbm.at[pl.ds(rs, 8), pl.ds(NFULL * CH, TAIL)],
            ctbuf, semt).wait()
        pltpu.make_async_copy(
            enc_hbm.at[pl.ds(rs, 8), pl.ds(NFULL * CH, TAIL)],
            etbuf, semu).wait()
        ms, mis, evs = _scan_group2(
            ctbuf, etbuf, NFULL * CH, NVREG_TAIL, ms, mis, evs)
        pltpu.make_async_copy(ctail_hbm.at[pl.ds(rs, 8)], c32, sem32).wait()
        pltpu.make_async_copy(etail_hbm.at[pl.ds(rs, 8)], e32, sem32).wait()
        ms, mis, evs = _scan_group2(c32, e32, VMAIN, 2, ms, mis, evs)

        for r in range(8):
            m, mi, ev = ms[r], mis[r], evs[r]
            for k in (8, 4, 2, 1):
                perm = lane ^ k
                om = m.at[perm].get(mode="promise_in_bounds")
                omi = mi.at[perm].get(mode="promise_in_bounds")
                oev = ev.at[perm].get(mode="promise_in_bounds")
                take = (om > m) | ((om == m) & (omi < mi))
                m = jnp.where(take, om, m)
                mi = jnp.where(take, omi, mi)
                ev = jnp.where(take, oev, ev)
            slot = 8 * g + r
            sel = lane == (slot % 16)
            accl[slot // 16] = jnp.where(sel, mi, accl[slot // 16])
            accv[slot // 16] = jnp.where(sel, ev, accv[slot // 16])

    for half in range(RPW // 16):
        idx_v[pl.ds(half * 16, 16)] = accl[half]
        vals_v[pl.ds(half * 16, 16)] = accv[half]

    pltpu.sync_copy(idx_v, l_hbm.at[pl.ds(row0, RPW)])
    pltpu.sync_copy(vals_v, v_hbm.at[pl.ds(row0, RPW)])


@functools.cache
def _sc_argmax_gather():
    return pl.kernel(
        _sc_body,
        mesh=plsc.VectorSubcoreMesh(core_axis_name="c", subcore_axis_name="s"),
        out_type=[
            jax.ShapeDtypeStruct((B,), jnp.int32),
            jax.ShapeDtypeStruct((B,), jnp.float32),
        ],
        scratch_types=[
            pltpu.VMEM((8, CH), jnp.float32),
            pltpu.VMEM((8, CH), jnp.float32),
            pltpu.VMEM((8, CH), jnp.float32),
            pltpu.VMEM((8, CH), jnp.float32),
            pltpu.VMEM((8, TAIL), jnp.float32),
            pltpu.VMEM((8, TAIL), jnp.float32),
            pltpu.VMEM((8, 32), jnp.float32),
            pltpu.VMEM((8, 32), jnp.float32),
            pltpu.VMEM((RPW,), jnp.int32),
            pltpu.VMEM((RPW,), jnp.float32),
            pltpu.SemaphoreType.DMA,
            pltpu.SemaphoreType.DMA,
            pltpu.SemaphoreType.DMA,
            pltpu.SemaphoreType.DMA,
            pltpu.SemaphoreType.DMA,
            pltpu.SemaphoreType.DMA,
            pltpu.SemaphoreType.DMA,
        ],
    )


def kernel(enc_score_p0, dec_scores, class_h_target, dec_target):
    ctail = class_h_target[:, VMAIN:]
    etail = enc_score_p0[:, VMAIN:]
    labels, v = _sc_argmax_gather()(class_h_target, enc_score_p0,
                                    ctail, etail)
    return _count_call(enc_score_p0, labels.reshape(B, 1), v.reshape(B, 1))
